# Initial kernel scaffold; baseline (speedup 1.0000x reference)
import jax
import jax.numpy as jnp
from jax.experimental import pallas as pl


def kernel(x, edge_index, W1, b1, W2, b2, Wq, bq):
    raise NotImplementedError("write your pallas kernel here")


# speedup vs baseline: 12.9980x; 12.9980x over previous
import jax
import jax.numpy as jnp
from jax import lax
from jax.experimental import pallas as pl
from jax.experimental.pallas import tpu as pltpu
from jax.experimental.pallas import tpu_sc as plsc

_N = 10000
_E = 320000
_DIN = 128
_H = 64
_NC = 2
_NS = 16
_NW = _NC * _NS
_EPW = 10240
_CH = 128
_NCHUNK = _EPW // _CH
_R = 10240
_RPT = _R // _NS

_mesh = plsc.VectorSubcoreMesh(core_axis_name="c", subcore_axis_name="s")


def _deg_body(dst_hbm, out_hbm, dst_v, ones_v, zrow_v, deg_sh):
    c = lax.axis_index("c")
    s = lax.axis_index("s")
    wid = s * _NC + c
    for i in range(_CH // 16):
        ones_v[pl.ds(i * 16, 16)] = jnp.ones((16,), jnp.float32)
    for i in range(_RPT // 16):
        zrow_v[pl.ds(i * 16, 16)] = jnp.zeros((16,), jnp.float32)
    pltpu.sync_copy(zrow_v, deg_sh.at[pl.ds(s * _RPT, _RPT)])
    plsc.subcore_barrier()
    pltpu.sync_copy(dst_hbm.at[wid], dst_v)

    def step(k, carry):
        pltpu.sync_copy(ones_v, deg_sh.at[dst_v.at[k]], add=True)
        return carry

    lax.fori_loop(0, _NCHUNK, step, 0)
    plsc.subcore_barrier()
    pltpu.sync_copy(deg_sh.at[pl.ds(s * _RPT, _RPT)],
                    out_hbm.at[c, pl.ds(s * _RPT, _RPT)])


_deg_call = pl.kernel(
    _deg_body,
    out_type=jax.ShapeDtypeStruct((_NC, _R), jnp.float32),
    mesh=_mesh,
    scratch_types=[
        pltpu.VMEM((_NCHUNK, _CH), jnp.int32),
        pltpu.VMEM((_CH,), jnp.float32),
        pltpu.VMEM((_RPT,), jnp.float32),
        pltpu.VMEM_SHARED((_R,), jnp.float32),
    ],
)


def _agg_body(hs_hbm, src_hbm, dst_hbm, out_hbm,
              src_v, dst_v, rows_v, zbuf, agg_sh, sem):
    c = lax.axis_index("c")
    s = lax.axis_index("s")
    wid = s * _NC + c
    for r in range(64):
        for j in range(_H // 16):
            zbuf[r, pl.ds(j * 16, 16)] = jnp.zeros((16,), jnp.float32)

    def zcp(j, carry):
        pltpu.sync_copy(zbuf, agg_sh.at[pl.ds(s * _RPT + j * 64, 64)])
        return carry

    lax.fori_loop(0, _RPT // 64, zcp, 0)
    plsc.subcore_barrier()
    pltpu.sync_copy(src_hbm.at[wid], src_v)
    pltpu.sync_copy(dst_hbm.at[wid], dst_v)

    def step(k, carry):
        pltpu.async_copy(hs_hbm.at[src_v.at[k]], rows_v, sem).wait()
        pltpu.sync_copy(rows_v, agg_sh.at[dst_v.at[k]], add=True)
        return carry

    lax.fori_loop(0, _NCHUNK, step, 0)
    plsc.subcore_barrier()
    pltpu.sync_copy(agg_sh.at[pl.ds(s * _RPT, _RPT)],
                    out_hbm.at[c, pl.ds(s * _RPT, _RPT)])


_agg_call = pl.kernel(
    _agg_body,
    out_type=jax.ShapeDtypeStruct((_NC, _R, _H), jnp.float32),
    mesh=_mesh,
    compiler_params=pltpu.CompilerParams(use_tc_tiling_on_sc=False),
    scratch_types=[
        pltpu.VMEM((_NCHUNK, _CH), jnp.int32),
        pltpu.VMEM((_NCHUNK, _CH), jnp.int32),
        pltpu.VMEM((_CH, _H), jnp.float32),
        pltpu.VMEM((64, _H), jnp.float32),
        pltpu.VMEM_SHARED((_R, _H), jnp.float32),
        pltpu.SemaphoreType.DMA,
    ],
)

_BM = 1000
_GRID = _N // _BM


def _k1_body(x_ref, w_ref, d0_ref, d1_ref, hs_ref, dinv_ref):
    deg = d0_ref[...] + d1_ref[...] + 1.0
    dinv = lax.rsqrt(deg)
    h = jnp.dot(x_ref[...], w_ref[...], preferred_element_type=jnp.float32)
    hs_ref[...] = h * dinv
    dinv_ref[...] = dinv


_k1_call = pl.pallas_call(
    _k1_body,
    grid=(_GRID,),
    in_specs=[
        pl.BlockSpec((_BM, _DIN), lambda i: (i, 0)),
        pl.BlockSpec((_DIN, _H), lambda i: (0, 0)),
        pl.BlockSpec((_BM, 1), lambda i: (i, 0)),
        pl.BlockSpec((_BM, 1), lambda i: (i, 0)),
    ],
    out_specs=[
        pl.BlockSpec((_BM, _H), lambda i: (i, 0)),
        pl.BlockSpec((_BM, 1), lambda i: (i, 0)),
    ],
    out_shape=[
        jax.ShapeDtypeStruct((_N, _H), jnp.float32),
        jax.ShapeDtypeStruct((_N, 1), jnp.float32),
    ],
)


def _k2_body(a0_ref, a1_ref, hs_ref, dinv_ref, b_ref, w_ref, out_ref):
    d = dinv_ref[...]
    t = jnp.maximum(d * (a0_ref[...] + a1_ref[...] + hs_ref[...]) + b_ref[...],
                    0.0)
    out_ref[...] = d * jnp.dot(t, w_ref[...], preferred_element_type=jnp.float32)


_k2_call = pl.pallas_call(
    _k2_body,
    grid=(_GRID,),
    in_specs=[
        pl.BlockSpec((_BM, _H), lambda i: (i, 0)),
        pl.BlockSpec((_BM, _H), lambda i: (i, 0)),
        pl.BlockSpec((_BM, _H), lambda i: (i, 0)),
        pl.BlockSpec((_BM, 1), lambda i: (i, 0)),
        pl.BlockSpec((1, _H), lambda i: (0, 0)),
        pl.BlockSpec((_H, _H), lambda i: (0, 0)),
    ],
    out_specs=pl.BlockSpec((_BM, _H), lambda i: (i, 0)),
    out_shape=jax.ShapeDtypeStruct((_N, _H), jnp.float32),
)


def _k3_body(a0_ref, a1_ref, hs_ref, dinv_ref, b_ref, wq_ref, bq_ref, out_ref):
    d = dinv_ref[...]
    t = jnp.maximum(d * (a0_ref[...] + a1_ref[...] + hs_ref[...]) + b_ref[...],
                    0.0)
    out_ref[...] = jnp.dot(t, wq_ref[...], preferred_element_type=jnp.float32) + bq_ref[...]


_k3_call = pl.pallas_call(
    _k3_body,
    grid=(_GRID,),
    in_specs=[
        pl.BlockSpec((_BM, _H), lambda i: (i, 0)),
        pl.BlockSpec((_BM, _H), lambda i: (i, 0)),
        pl.BlockSpec((_BM, _H), lambda i: (i, 0)),
        pl.BlockSpec((_BM, 1), lambda i: (i, 0)),
        pl.BlockSpec((1, _H), lambda i: (0, 0)),
        pl.BlockSpec((_H, 1), lambda i: (0, 0)),
        pl.BlockSpec((1, 1), lambda i: (0, 0)),
    ],
    out_specs=pl.BlockSpec((_BM, 1), lambda i: (i, 0)),
    out_shape=jax.ShapeDtypeStruct((_N, 1), jnp.float32),
)


def kernel(x, edge_index, W1, b1, W2, b2, Wq, bq):
    src = edge_index[0].astype(jnp.int32)
    dst = edge_index[1].astype(jnp.int32)
    pad = _NW * _EPW - _E
    src3 = jnp.concatenate([src, jnp.zeros((pad,), jnp.int32)])
    src3 = src3.reshape(_NW, _NCHUNK, _CH)
    dst3 = jnp.concatenate([dst, jnp.full((pad,), _N, jnp.int32)])
    dst3 = dst3.reshape(_NW, _NCHUNK, _CH)

    degp = _deg_call(dst3)
    d0 = degp[0, :_N, None]
    d1 = degp[1, :_N, None]
    hs1, dinv = _k1_call(x, W1, d0, d1)

    aggp = _agg_call(hs1, src3, dst3)
    hs2 = _k2_call(aggp[0, :_N], aggp[1, :_N], hs1, dinv,
                   b1.reshape(1, _H), W2)

    aggp2 = _agg_call(hs2, src3, dst3)
    q = _k3_call(aggp2[0, :_N], aggp2[1, :_N], hs2, dinv,
                 b2.reshape(1, _H), Wq, bq.reshape(1, 1))
    return q[:, 0]


# speedup vs baseline: 15.1210x; 1.1633x over previous
import jax
import jax.numpy as jnp
from jax import lax
from jax.experimental import pallas as pl
from jax.experimental.pallas import tpu as pltpu
from jax.experimental.pallas import tpu_sc as plsc

_N = 10000
_E = 320000
_DIN = 128
_H = 64
_NC = 2
_NS = 16
_NW = _NC * _NS
_EPW = 10240
_CH = 128
_NCHUNK = _EPW // _CH
_R = 10240
_RPT = _R // _NS

_mesh = plsc.VectorSubcoreMesh(core_axis_name="c", subcore_axis_name="s")


def _deg_body(dst_hbm, out_hbm, dst_v, ones_v, zrow_v, deg_sh):
    c = lax.axis_index("c")
    s = lax.axis_index("s")
    wid = s * _NC + c
    for i in range(_CH // 16):
        ones_v[pl.ds(i * 16, 16)] = jnp.ones((16,), jnp.float32)
    for i in range(_RPT // 16):
        zrow_v[pl.ds(i * 16, 16)] = jnp.zeros((16,), jnp.float32)
    pltpu.sync_copy(zrow_v, deg_sh.at[pl.ds(s * _RPT, _RPT)])
    plsc.subcore_barrier()
    pltpu.sync_copy(dst_hbm.at[wid], dst_v)

    def step(k, carry):
        pltpu.sync_copy(ones_v, deg_sh.at[dst_v.at[k]], add=True)
        return carry

    lax.fori_loop(0, _NCHUNK, step, 0)
    plsc.subcore_barrier()
    pltpu.sync_copy(deg_sh.at[pl.ds(s * _RPT, _RPT)],
                    out_hbm.at[c, pl.ds(s * _RPT, _RPT)])


_deg_call = pl.kernel(
    _deg_body,
    out_type=jax.ShapeDtypeStruct((_NC, _R), jnp.float32),
    mesh=_mesh,
    scratch_types=[
        pltpu.VMEM((_NCHUNK, _CH), jnp.int32),
        pltpu.VMEM((_CH,), jnp.float32),
        pltpu.VMEM((_RPT,), jnp.float32),
        pltpu.VMEM_SHARED((_R,), jnp.float32),
    ],
)


def _agg_body(hs_hbm, src_hbm, dst_hbm, out_hbm,
              src_v, dst_v, rows_a, rows_b, zbuf, agg_sh, sem_a, sem_b):
    c = lax.axis_index("c")
    s = lax.axis_index("s")
    wid = s * _NC + c
    for r in range(64):
        for j in range(_H // 16):
            zbuf[r, pl.ds(j * 16, 16)] = jnp.zeros((16,), jnp.float32)

    def zcp(j, carry):
        pltpu.sync_copy(zbuf, agg_sh.at[pl.ds(s * _RPT + j * 64, 64)])
        return carry

    lax.fori_loop(0, _RPT // 64, zcp, 0)
    plsc.subcore_barrier()
    pltpu.sync_copy(src_hbm.at[wid], src_v)
    pltpu.sync_copy(dst_hbm.at[wid], dst_v)

    pltpu.async_copy(hs_hbm.at[src_v.at[0]], rows_a, sem_a)

    def step2(g, carry):
        k0 = 2 * g
        pltpu.async_copy(hs_hbm.at[src_v.at[k0 + 1]], rows_b, sem_b)
        pltpu.make_async_copy(hs_hbm.at[src_v.at[k0]], rows_a, sem_a).wait()
        pltpu.sync_copy(rows_a, agg_sh.at[dst_v.at[k0]], add=True)

        @pl.when(g + 1 < _NCHUNK // 2)
        def _():
            pltpu.async_copy(hs_hbm.at[src_v.at[k0 + 2]], rows_a, sem_a)

        pltpu.make_async_copy(hs_hbm.at[src_v.at[k0 + 1]], rows_b, sem_b).wait()
        pltpu.sync_copy(rows_b, agg_sh.at[dst_v.at[k0 + 1]], add=True)
        return carry

    lax.fori_loop(0, _NCHUNK // 2, step2, 0)
    plsc.subcore_barrier()
    pltpu.sync_copy(agg_sh.at[pl.ds(s * _RPT, _RPT)],
                    out_hbm.at[c, pl.ds(s * _RPT, _RPT)])


_agg_call = pl.kernel(
    _agg_body,
    out_type=jax.ShapeDtypeStruct((_NC, _R, _H), jnp.float32),
    mesh=_mesh,
    compiler_params=pltpu.CompilerParams(use_tc_tiling_on_sc=False),
    scratch_types=[
        pltpu.VMEM((_NCHUNK, _CH), jnp.int32),
        pltpu.VMEM((_NCHUNK, _CH), jnp.int32),
        pltpu.VMEM((_CH, _H), jnp.float32),
        pltpu.VMEM((_CH, _H), jnp.float32),
        pltpu.VMEM((64, _H), jnp.float32),
        pltpu.VMEM_SHARED((_R, _H), jnp.float32),
        pltpu.SemaphoreType.DMA,
        pltpu.SemaphoreType.DMA,
    ],
)

_BM = 1000
_GRID = _N // _BM


def _k1_body(x_ref, w_ref, d0_ref, d1_ref, hs_ref, dinv_ref):
    deg = d0_ref[...] + d1_ref[...] + 1.0
    dinv = lax.rsqrt(deg)
    h = jnp.dot(x_ref[...], w_ref[...], preferred_element_type=jnp.float32)
    hs_ref[...] = h * dinv
    dinv_ref[...] = dinv


_k1_call = pl.pallas_call(
    _k1_body,
    grid=(_GRID,),
    in_specs=[
        pl.BlockSpec((_BM, _DIN), lambda i: (i, 0)),
        pl.BlockSpec((_DIN, _H), lambda i: (0, 0)),
        pl.BlockSpec((_BM, 1), lambda i: (i, 0)),
        pl.BlockSpec((_BM, 1), lambda i: (i, 0)),
    ],
    out_specs=[
        pl.BlockSpec((_BM, _H), lambda i: (i, 0)),
        pl.BlockSpec((_BM, 1), lambda i: (i, 0)),
    ],
    out_shape=[
        jax.ShapeDtypeStruct((_N, _H), jnp.float32),
        jax.ShapeDtypeStruct((_N, 1), jnp.float32),
    ],
)


def _k2_body(a0_ref, a1_ref, hs_ref, dinv_ref, b_ref, w_ref, out_ref):
    d = dinv_ref[...]
    t = jnp.maximum(d * (a0_ref[...] + a1_ref[...] + hs_ref[...]) + b_ref[...],
                    0.0)
    out_ref[...] = d * jnp.dot(t, w_ref[...], preferred_element_type=jnp.float32)


_k2_call = pl.pallas_call(
    _k2_body,
    grid=(_GRID,),
    in_specs=[
        pl.BlockSpec((_BM, _H), lambda i: (i, 0)),
        pl.BlockSpec((_BM, _H), lambda i: (i, 0)),
        pl.BlockSpec((_BM, _H), lambda i: (i, 0)),
        pl.BlockSpec((_BM, 1), lambda i: (i, 0)),
        pl.BlockSpec((1, _H), lambda i: (0, 0)),
        pl.BlockSpec((_H, _H), lambda i: (0, 0)),
    ],
    out_specs=pl.BlockSpec((_BM, _H), lambda i: (i, 0)),
    out_shape=jax.ShapeDtypeStruct((_N, _H), jnp.float32),
)


def _k3_body(a0_ref, a1_ref, hs_ref, dinv_ref, b_ref, wq_ref, bq_ref, out_ref):
    d = dinv_ref[...]
    t = jnp.maximum(d * (a0_ref[...] + a1_ref[...] + hs_ref[...]) + b_ref[...],
                    0.0)
    out_ref[...] = jnp.dot(t, wq_ref[...], preferred_element_type=jnp.float32) + bq_ref[...]


_k3_call = pl.pallas_call(
    _k3_body,
    grid=(_GRID,),
    in_specs=[
        pl.BlockSpec((_BM, _H), lambda i: (i, 0)),
        pl.BlockSpec((_BM, _H), lambda i: (i, 0)),
        pl.BlockSpec((_BM, _H), lambda i: (i, 0)),
        pl.BlockSpec((_BM, 1), lambda i: (i, 0)),
        pl.BlockSpec((1, _H), lambda i: (0, 0)),
        pl.BlockSpec((_H, 1), lambda i: (0, 0)),
        pl.BlockSpec((1, 1), lambda i: (0, 0)),
    ],
    out_specs=pl.BlockSpec((_BM, 1), lambda i: (i, 0)),
    out_shape=jax.ShapeDtypeStruct((_N, 1), jnp.float32),
)


def kernel(x, edge_index, W1, b1, W2, b2, Wq, bq):
    src = edge_index[0].astype(jnp.int32)
    dst = edge_index[1].astype(jnp.int32)
    pad = _NW * _EPW - _E
    src3 = jnp.concatenate([src, jnp.zeros((pad,), jnp.int32)])
    src3 = src3.reshape(_NW, _NCHUNK, _CH)
    dst3 = jnp.concatenate([dst, jnp.full((pad,), _N, jnp.int32)])
    dst3 = dst3.reshape(_NW, _NCHUNK, _CH)

    degp = _deg_call(dst3)
    d0 = degp[0, :_N, None]
    d1 = degp[1, :_N, None]
    hs1, dinv = _k1_call(x, W1, d0, d1)

    aggp = _agg_call(hs1, src3, dst3)
    hs2 = _k2_call(aggp[0, :_N], aggp[1, :_N], hs1, dinv,
                   b1.reshape(1, _H), W2)

    aggp2 = _agg_call(hs2, src3, dst3)
    q = _k3_call(aggp2[0, :_N], aggp2[1, :_N], hs2, dinv,
                 b2.reshape(1, _H), Wq, bq.reshape(1, 1))
    return q[:, 0]
